# baseline (speedup 1.0000x reference)
import functools

import jax
import jax.numpy as jnp
from jax import lax
from jax.experimental import pallas as pl
from jax.experimental.pallas import tpu as pltpu
from jax.experimental.pallas import tpu_sc as plsc

D = 128
NC, NS = 2, 16
NW = NC * NS
CHUNK = 128


def _tables_body(k_ref, p_ref, q_ref, w1_ref, b1_ref, a_ref, b_ref):
    x = k_ref[...] + p_ref[...]
    w1 = w1_ref[...]
    a_ref[...] = jnp.dot(x, w1, preferred_element_type=jnp.float32) + b1_ref[...]
    b_ref[...] = jnp.dot(q_ref[...], w1, preferred_element_type=jnp.float32)


def _node_tables(K_h, Q_h, P_e, W1, b1):
    n = K_h.shape[0]
    blk = 1000
    row_spec = pl.BlockSpec((blk, D), lambda i: (i, 0))
    return pl.pallas_call(
        _tables_body,
        grid=(n // blk,),
        in_specs=[row_spec, row_spec, row_spec,
                  pl.BlockSpec((D, D), lambda i: (0, 0)),
                  pl.BlockSpec((1, D), lambda i: (0, 0))],
        out_specs=[row_spec, row_spec],
        out_shape=[jax.ShapeDtypeStruct((n, D), jnp.float32),
                   jax.ShapeDtypeStruct((n, D), jnp.float32)],
    )(K_h, P_e, Q_h, W1, b1.reshape(1, D))


def _gather_rows(A, B, src, dst, e_pad):
    per_w = e_pad // NW
    n_chunks = per_w // CHUNK
    assert n_chunks % 2 == 0 and n_chunks >= 6
    mesh = plsc.VectorSubcoreMesh(core_axis_name="c", subcore_axis_name="s",
                                  num_cores=NC, num_subcores=NS)

    @functools.partial(
        pl.kernel,
        out_type=[jax.ShapeDtypeStruct((e_pad, D), jnp.float32),
                  jax.ShapeDtypeStruct((e_pad, D), jnp.float32)],
        mesh=mesh,
        compiler_params=pltpu.CompilerParams(needs_layout_passes=False),
        scratch_types=[
            pltpu.VMEM((CHUNK,), jnp.int32),
            pltpu.VMEM((CHUNK,), jnp.int32),
            pltpu.VMEM((CHUNK,), jnp.int32),
            pltpu.VMEM((CHUNK,), jnp.int32),
            pltpu.VMEM((CHUNK, D), jnp.float32),
            pltpu.VMEM((CHUNK, D), jnp.float32),
            pltpu.VMEM((CHUNK, D), jnp.float32),
            pltpu.VMEM((CHUNK, D), jnp.float32),
            pltpu.SemaphoreType.DMA,
            pltpu.SemaphoreType.DMA,
            pltpu.SemaphoreType.DMA,
            pltpu.SemaphoreType.DMA,
            pltpu.SemaphoreType.DMA,
            pltpu.SemaphoreType.DMA,
        ],
    )
    def k(a_hbm, b_hbm, src_hbm, dst_hbm, ga_hbm, gb_hbm,
          idx_s0, idx_d0, idx_s1, idx_d1, bufa0, bufb0, bufa1, bufb1,
          sem_i0, sem_i1, sem_g0, sem_g1, sem_o0, sem_o1):
        wid = lax.axis_index("s") * NC + lax.axis_index("c")
        base_w = wid * per_w
        idx_s, idx_d = (idx_s0, idx_s1), (idx_d0, idx_d1)
        bufa, bufb = (bufa0, bufa1), (bufb0, bufb1)
        sem_i, sem_g, sem_o = (sem_i0, sem_i1), (sem_g0, sem_g1), (sem_o0, sem_o1)

        def idx_cp(c, b):
            base = base_w + c * CHUNK
            return (pltpu.make_async_copy(src_hbm.at[pl.ds(base, CHUNK)],
                                          idx_s[b], sem_i[b]),
                    pltpu.make_async_copy(dst_hbm.at[pl.ds(base, CHUNK)],
                                          idx_d[b], sem_i[b]))

        def gat_cp(b):
            return (pltpu.make_async_copy(a_hbm.at[idx_s[b]], bufa[b], sem_g[b]),
                    pltpu.make_async_copy(b_hbm.at[idx_d[b]], bufb[b], sem_g[b]))

        def out_cp(c, b):
            base = base_w + c * CHUNK
            return (pltpu.make_async_copy(bufa[b], ga_hbm.at[pl.ds(base, CHUNK)],
                                          sem_o[b]),
                    pltpu.make_async_copy(bufb[b], gb_hbm.at[pl.ds(base, CHUNK)],
                                          sem_o[b]))

        def start(cps):
            for cp in cps:
                cp.start()

        def wait(cps):
            for cp in cps:
                cp.wait()

        def step(c, b, first, prefetch):
            wait(idx_cp(c + 1, 1 - b))
            if not first:
                wait(out_cp(c - 1, 1 - b))
            start(gat_cp(1 - b))
            wait(gat_cp(b))
            start(out_cp(c, b))
            if prefetch:
                start(idx_cp(c + 2, b))

        start(idx_cp(0, 0))
        start(idx_cp(1, 1))
        wait(idx_cp(0, 0))
        start(gat_cp(0))
        step(0, 0, first=True, prefetch=True)
        step(1, 1, first=False, prefetch=True)

        def body(t, carry):
            step(2 * t, 0, first=False, prefetch=True)
            step(2 * t + 1, 1, first=False, prefetch=True)
            return carry

        lax.fori_loop(1, n_chunks // 2 - 1, body, 0)

        step(n_chunks - 2, 0, first=False, prefetch=False)
        wait(gat_cp(1))
        start(out_cp(n_chunks - 1, 1))
        wait(out_cp(n_chunks - 2, 0))
        wait(out_cp(n_chunks - 1, 1))

    return k(A, B, src, dst)


def _score_body(ga_ref, gb_ref, w2_ref, b2_ref, out_ref):
    h = jnp.maximum(ga_ref[...] - gb_ref[...], 0.0)
    out_ref[...] = (jnp.dot(h, w2_ref[...], preferred_element_type=jnp.float32)
                    + b2_ref[...])


def _edge_scores(GA, GB, W2, b2, e_pad):
    blk = 2048
    row_spec = pl.BlockSpec((blk, D), lambda i: (i, 0))
    return pl.pallas_call(
        _score_body,
        grid=(e_pad // blk,),
        in_specs=[row_spec, row_spec,
                  pl.BlockSpec((D, 1), lambda i: (0, 0)),
                  pl.BlockSpec((1, 1), lambda i: (0, 0))],
        out_specs=pl.BlockSpec((blk, 1), lambda i: (i, 0)),
        out_shape=jax.ShapeDtypeStruct((e_pad, 1), jnp.float32),
    )(GA, GB, W2, b2.reshape(1, 1))


def kernel(K_h, Q_h, P_e, edge_index, W1, b1, W2, b2):
    n_edges = edge_index.shape[1]
    A, B = _node_tables(K_h, Q_h, P_e, W1, b1)

    grain = NW * CHUNK * 2
    e_pad = ((n_edges + grain - 1) // grain) * grain
    pad = e_pad - n_edges
    src = jnp.concatenate([edge_index[0], jnp.zeros((pad,), jnp.int32)])
    dst = jnp.concatenate([edge_index[1], jnp.zeros((pad,), jnp.int32)])

    GA, GB = _gather_rows(A, B, src, dst, e_pad)
    scores = _edge_scores(GA, GB, W2, b2, e_pad)
    return scores[:n_edges]
